# baseline (speedup 1.0000x reference)
import jax
import jax.numpy as jnp
from jax import lax
from jax.experimental import pallas as pl
from jax.experimental.pallas import tpu as pltpu
from jax.experimental.pallas import tpu_sc as plsc

_V = 1000
_D = 64
_DP = 128
_B = 16384
_NC = 2
_NS = 16
_NW = _NC * _NS
_BPW = _B // _NW
_CHUNK = 128
_NCHUNK = _BPW // _CHUNK


def _gather_body(idx_hbm, table_hbm, out_hbm, idx_v, rows_v, rowsT_v, sem, sem2):
    wid = lax.axis_index("s") * _NC + lax.axis_index("c")
    base = wid * _BPW
    idx_copies = [
        pltpu.async_copy(
            idx_hbm.at[pl.ds(base + j * _CHUNK, _CHUNK)],
            idx_v.at[pl.ds(j * _CHUNK, _CHUNK)],
            sem2,
        )
        for j in range(_NCHUNK)
    ]
    copies = []
    for j in range(_NCHUNK):
        idx_copies[j].wait()
        copies.append(
            pltpu.async_copy(
                table_hbm.at[idx_v.at[pl.ds(j * _CHUNK, _CHUNK)]],
                rows_v.at[pl.ds(j * _CHUNK, _CHUNK)],
                sem,
            )
        )
    iota16 = lax.iota(jnp.int32, 16)
    half = _D // 2
    dvecs = [half + q * 16 + iota16 for q in range((_D - half) // 16)]
    dhis = [dv // 8 for dv in dvecs]
    dlos = [dv % 8 for dv in dvecs]
    wbs = []
    for j in range(_NCHUNK):
        copies[j].wait()
        jv = jnp.zeros((16,), jnp.int32) + j

        @plsc.parallel_loop(jnp.int32(0), jnp.int32(_CHUNK), jnp.int32(1), unroll=4)
        def _tr(i, j=j, jv=jv):
            d = i // 4
            d_hi = d // 8
            d_lo = d % 8
            col = jnp.zeros((16,), jnp.int32) + d
            for gg in range(2):
                g = (i % 4) * 2 + gg
                val = plsc.load_gather(
                    rows_v, [iota16 + j * _CHUNK + g * 16, col]
                )
                rowsT_v[d_hi, jnp.int32(j), d_lo, pl.ds(g * 16, 16)] = val
            bv = jnp.zeros((16,), jnp.int32) + i
            for q in range((_D - half) // 16):
                row = rows_v[i + j * _CHUNK, pl.ds(half + q * 16, 16)]
                plsc.store_scatter(rowsT_v, [dhis[q], jv, dlos[q], bv], row)

        wbs.append(
            pltpu.async_copy(
                rowsT_v.at[:, pl.ds(j, 1), :, :],
                out_hbm.at[:, pl.ds(wid * _NCHUNK + j, 1), :, :],
                sem2,
            )
        )
    for w in wbs:
        w.wait()


def kernel(x, table):
    idx = x.astype(jnp.int32)
    f = pl.kernel(
        _gather_body,
        out_type=jax.ShapeDtypeStruct((_D // 8, _B // _CHUNK, 8, _CHUNK), jnp.float32),
        mesh=plsc.VectorSubcoreMesh(core_axis_name="c", subcore_axis_name="s"),
        scratch_types=[
            pltpu.VMEM((_BPW,), jnp.int32),
            pltpu.VMEM((_BPW, _D), jnp.float32),
            pltpu.VMEM((_D // 8, _NCHUNK, 8, _CHUNK), jnp.float32),
            pltpu.SemaphoreType.DMA,
            pltpu.SemaphoreType.DMA,
        ],
        compiler_params=pltpu.CompilerParams(
            use_tc_tiling_on_sc=False, needs_layout_passes=False
        ),
    )
    return f(idx, table).transpose(1, 3, 0, 2).reshape(_B, _D)
